# baseline (speedup 1.0000x reference)
import functools

import jax
import jax.numpy as jnp
from jax import lax
from jax.experimental import pallas as pl
from jax.experimental.pallas import tpu as pltpu
from jax.experimental.pallas import tpu_sc as plsc

N = 10000
D = 128
E = 80000
R = 4
NB = 2

_TILES = 16
_CORES = 2
_REL_PER_CORE = R // _CORES
CH = 128
EPT = E // _TILES
NCH = -(-EPT // CH)
SLOTS = NCH * CH
TAIL = (EPT // 16) * 16
_NBUF = 4
ACC_ROWS = 10240
STRIPE = ACC_ROWS // _TILES
BN = 2000


def _sc_segment_sums(x_bf, e0, e1, e2, e3, zeros_stripe):
    mesh = plsc.VectorSubcoreMesh(core_axis_name="c", subcore_axis_name="s")

    @functools.partial(
        pl.kernel,
        mesh=mesh,
        out_type=jax.ShapeDtypeStruct((R * _TILES, STRIPE, D), jnp.bfloat16),
        compiler_params=pltpu.CompilerParams(use_tc_tiling_on_sc=False),
        scratch_types=[
            pltpu.VMEM((SLOTS,), jnp.int32),
            pltpu.VMEM((SLOTS,), jnp.int32),
            pltpu.VMEM((_NBUF, CH, D), jnp.bfloat16),
            pltpu.VMEM_SHARED((ACC_ROWS, D), jnp.bfloat16),
            pltpu.VMEM_SHARED((N, D), jnp.bfloat16),
        ]
        + [pltpu.SemaphoreType.DMA] * (2 * _NBUF),
    )
    def body(x_hbm, e0_hbm, e1_hbm, e2_hbm, e3_hbm, zero_hbm, out_hbm, src_v, dst_v, rows_v, acc, xs, *sems):
        e_refs = (e0_hbm, e1_hbm, e2_hbm, e3_hbm)
        gsem = sems[:_NBUF]
        ssem = sems[_NBUF:]
        c = lax.axis_index("c")
        s = lax.axis_index("s")

        def src_idx(i):
            return src_v.at[pl.ds(i * CH, CH)]

        def dst_idx(i):
            return dst_v.at[pl.ds(i * CH, CH)]

        def wait_gather(i, b):
            pltpu.make_async_copy(xs.at[src_idx(i)], rows_v.at[b], gsem[b]).wait()

        def wait_scatter(i, b):
            pltpu.make_async_copy(rows_v.at[b], acc.at[dst_idx(i)], ssem[b]).wait()

        pltpu.sync_copy(
            x_hbm.at[pl.ds(s * (N // _TILES), N // _TILES)],
            xs.at[pl.ds(s * (N // _TILES), N // _TILES)],
        )

        for j in range(_REL_PER_CORE):
            rel = c * _REL_PER_CORE + j
            blk = rel * _TILES + s
            pltpu.sync_copy(zero_hbm, acc.at[pl.ds(s * STRIPE, STRIPE)])
            for q in range((SLOTS - TAIL) // 16):
                src_v[pl.ds(TAIL + 16 * q, 16)] = jnp.zeros((16,), jnp.int32)
                dst_v[pl.ds(TAIL + 16 * q, 16)] = jnp.full((16,), N, jnp.int32)
            for cc in range(_CORES):
                @pl.when(c == cc)
                def _stage_idx():
                    e_hbm = e_refs[cc * _REL_PER_CORE + j]
                    pltpu.sync_copy(e_hbm.at[0].at[pl.ds(s * EPT, EPT)], src_v.at[pl.ds(0, EPT)])
                    pltpu.sync_copy(e_hbm.at[1].at[pl.ds(s * EPT, EPT)], dst_v.at[pl.ds(0, EPT)])
            plsc.subcore_barrier()

            for b in range(_NBUF - 1):
                pltpu.async_copy(xs.at[src_idx(b)], rows_v.at[b], gsem[b])

            def step(k, carry):
                for b in range(_NBUF):
                    i = _NBUF * k + b
                    wait_gather(i, b)
                    pltpu.async_copy(rows_v.at[b], acc.at[dst_idx(i)], ssem[b], add=True)
                    b2 = (b + _NBUF - 1) % _NBUF

                    @pl.when(i + _NBUF - 1 < NCH)
                    def _issue():
                        if b == 0:
                            @pl.when(k > 0)
                            def _w():
                                wait_scatter(i - 1, b2)
                        else:
                            wait_scatter(i - 1, b2)
                        pltpu.async_copy(xs.at[src_idx(i + _NBUF - 1)], rows_v.at[b2], gsem[b2])
                return carry

            lax.fori_loop(0, NCH // _NBUF, step, 0)
            for b in range(_NBUF):
                wait_scatter(NCH - _NBUF + b, b)
            plsc.subcore_barrier()
            pltpu.sync_copy(acc.at[pl.ds(s * STRIPE, STRIPE)], out_hbm.at[blk])

    return body(x_bf, e0, e1, e2, e3, zeros_stripe)


def _combine_body(g_ref, x_ref, att_ref, basis_ref, root_ref, o_ref):
    w = jnp.dot(
        att_ref[...] * 0.25,
        basis_ref[...].reshape(NB, D * D),
        preferred_element_type=jnp.float32,
    ).reshape(R, D, D).astype(jnp.bfloat16)
    acc = jnp.dot(x_ref[...], root_ref[...], preferred_element_type=jnp.float32)
    for r in range(R):
        acc = acc + jnp.dot(g_ref[r], w[r], preferred_element_type=jnp.float32)
    o_ref[...] = acc


def _combine(g, x, att, basis, root):
    return pl.pallas_call(
        _combine_body,
        grid=(N // BN,),
        in_specs=[
            pl.BlockSpec((R, BN, D), lambda i: (0, i, 0)),
            pl.BlockSpec((BN, D), lambda i: (i, 0)),
            pl.BlockSpec((R, NB), lambda i: (0, 0)),
            pl.BlockSpec((NB, D, D), lambda i: (0, 0, 0)),
            pl.BlockSpec((D, D), lambda i: (0, 0)),
        ],
        out_specs=pl.BlockSpec((BN, D), lambda i: (i, 0)),
        out_shape=jax.ShapeDtypeStruct((N, D), jnp.float32),
    )(g, x, att, basis, root)


def kernel(x, edge_index_0, edge_index_1, edge_index_2, edge_index_3, dest, att, basis, root):
    del dest
    zeros_stripe = jnp.zeros((STRIPE, D), jnp.bfloat16)

    g = _sc_segment_sums(
        x.astype(jnp.bfloat16),
        edge_index_0, edge_index_1, edge_index_2, edge_index_3,
        zeros_stripe,
    )
    g = g.reshape(R, ACC_ROWS, D)
    return _combine(g, x, att, basis, root)


# speedup vs baseline: 1.0028x; 1.0028x over previous
import functools

import jax
import jax.numpy as jnp
from jax import lax
from jax.experimental import pallas as pl
from jax.experimental.pallas import tpu as pltpu
from jax.experimental.pallas import tpu_sc as plsc

N = 10000
D = 128
E = 80000
R = 4
NB = 2

_TILES = 16
_CORES = 2
_REL_PER_CORE = R // _CORES
CH = 128
EPT = E // _TILES
NCH = -(-EPT // CH)
SLOTS = NCH * CH
TAIL = (EPT // 16) * 16
_NBUF = 4
ACC_ROWS = 10240
STRIPE = ACC_ROWS // _TILES
BN = 2000


def _sc_segment_sums(x_bf, e0, e1, e2, e3, zeros_stripe):
    mesh = plsc.VectorSubcoreMesh(core_axis_name="c", subcore_axis_name="s")

    @functools.partial(
        pl.kernel,
        mesh=mesh,
        out_type=jax.ShapeDtypeStruct((R * _TILES, STRIPE, D), jnp.bfloat16),
        compiler_params=pltpu.CompilerParams(use_tc_tiling_on_sc=False),
        scratch_types=[
            pltpu.VMEM((SLOTS,), jnp.int32),
            pltpu.VMEM((SLOTS,), jnp.int32),
            pltpu.VMEM((_NBUF, CH, D), jnp.bfloat16),
            pltpu.VMEM_SHARED((ACC_ROWS, D), jnp.bfloat16),
            pltpu.VMEM_SHARED((N, D), jnp.bfloat16),
        ]
        + [pltpu.SemaphoreType.DMA] * (2 * _NBUF),
    )
    def body(x_hbm, e0_hbm, e1_hbm, e2_hbm, e3_hbm, zero_hbm, out_hbm, src_v, dst_v, rows_v, acc, xs, *sems):
        e_refs = (e0_hbm, e1_hbm, e2_hbm, e3_hbm)
        gsem = sems[:_NBUF]
        ssem = sems[_NBUF:]
        c = lax.axis_index("c")
        s = lax.axis_index("s")

        def src_idx(i):
            return src_v.at[pl.ds(i * CH, CH)]

        def dst_idx(i):
            return dst_v.at[pl.ds(i * CH, CH)]

        def wait_gather(i, b):
            pltpu.make_async_copy(xs.at[src_idx(i)], rows_v.at[b], gsem[b]).wait()

        def wait_scatter(i, b):
            pltpu.make_async_copy(rows_v.at[b], acc.at[dst_idx(i)], ssem[b]).wait()

        pltpu.sync_copy(
            x_hbm.at[pl.ds(s * (N // _TILES), N // _TILES)],
            xs.at[pl.ds(s * (N // _TILES), N // _TILES)],
        )

        for j in range(_REL_PER_CORE):
            rel = c * _REL_PER_CORE + j
            blk = rel * _TILES + s
            pltpu.sync_copy(zero_hbm, acc.at[pl.ds(s * STRIPE, STRIPE)])
            for q in range((SLOTS - TAIL) // 16):
                src_v[pl.ds(TAIL + 16 * q, 16)] = jnp.zeros((16,), jnp.int32)
                dst_v[pl.ds(TAIL + 16 * q, 16)] = jnp.full((16,), N, jnp.int32)
            for cc in range(_CORES):
                @pl.when(c == cc)
                def _stage_idx():
                    e_hbm = e_refs[cc * _REL_PER_CORE + j]
                    pltpu.sync_copy(e_hbm.at[0].at[pl.ds(s * EPT, EPT)], src_v.at[pl.ds(0, EPT)])
                    pltpu.sync_copy(e_hbm.at[1].at[pl.ds(s * EPT, EPT)], dst_v.at[pl.ds(0, EPT)])
            plsc.subcore_barrier()

            for b in range(_NBUF - 1):
                pltpu.async_copy(xs.at[src_idx(b)], rows_v.at[b], gsem[b])

            def step(k, carry):
                for b in range(_NBUF):
                    i = _NBUF * k + b
                    wait_gather(i, b)
                    pltpu.async_copy(rows_v.at[b], acc.at[dst_idx(i)], ssem[b], add=True)
                    b2 = (b + _NBUF - 1) % _NBUF

                    @pl.when(i + _NBUF - 1 < NCH)
                    def _issue():
                        if b == 0:
                            @pl.when(k > 0)
                            def _w():
                                wait_scatter(i - 1, b2)
                        else:
                            wait_scatter(i - 1, b2)
                        pltpu.async_copy(xs.at[src_idx(i + _NBUF - 1)], rows_v.at[b2], gsem[b2])
                return carry

            lax.fori_loop(0, NCH // _NBUF, step, 0)
            for b in range(_NBUF):
                wait_scatter(NCH - _NBUF + b, b)
            plsc.subcore_barrier()
            pltpu.sync_copy(acc.at[pl.ds(s * STRIPE, STRIPE)], out_hbm.at[blk])

    return body(x_bf, e0, e1, e2, e3, zeros_stripe)


def _root_mm_body(x_ref, root_ref, o_ref):
    o_ref[...] = jnp.dot(x_ref[...], root_ref[...], preferred_element_type=jnp.float32)


def _root_mm(x, root):
    return pl.pallas_call(
        _root_mm_body,
        grid=(N // BN,),
        in_specs=[
            pl.BlockSpec((BN, D), lambda i: (i, 0)),
            pl.BlockSpec((D, D), lambda i: (0, 0)),
        ],
        out_specs=pl.BlockSpec((BN, D), lambda i: (i, 0)),
        out_shape=jax.ShapeDtypeStruct((N, D), jnp.float32),
    )(x, root)


def _combine_body(g_ref, p_ref, att_ref, basis_ref, o_ref):
    acc = p_ref[...]
    gf = g_ref[...].astype(jnp.float32)
    g0, g1, g2, g3 = gf[0], gf[1], gf[2], gf[3]
    m0 = att_ref[0, 0] * g0 + att_ref[1, 0] * g1 + att_ref[2, 0] * g2 + att_ref[3, 0] * g3
    m1 = att_ref[0, 1] * g0 + att_ref[1, 1] * g1 + att_ref[2, 1] * g2 + att_ref[3, 1] * g3
    acc = acc + 0.25 * (
        jnp.dot(m0, basis_ref[0], preferred_element_type=jnp.float32)
        + jnp.dot(m1, basis_ref[1], preferred_element_type=jnp.float32)
    )
    o_ref[...] = acc


def _combine(g, p, att, basis):
    return pl.pallas_call(
        _combine_body,
        grid=(N // BN,),
        in_specs=[
            pl.BlockSpec((R, BN, D), lambda i: (0, i, 0)),
            pl.BlockSpec((BN, D), lambda i: (i, 0)),
            pl.BlockSpec(memory_space=pltpu.SMEM),
            pl.BlockSpec((NB, D, D), lambda i: (0, 0, 0)),
        ],
        out_specs=pl.BlockSpec((BN, D), lambda i: (i, 0)),
        out_shape=jax.ShapeDtypeStruct((N, D), jnp.float32),
    )(g, p, att, basis)


def kernel(x, edge_index_0, edge_index_1, edge_index_2, edge_index_3, dest, att, basis, root):
    del dest
    zeros_stripe = jnp.zeros((STRIPE, D), jnp.bfloat16)

    p = _root_mm(x, root)
    g = _sc_segment_sums(
        x.astype(jnp.bfloat16),
        edge_index_0, edge_index_1, edge_index_2, edge_index_3,
        zeros_stripe,
    )
    g = g.reshape(R, ACC_ROWS, D)
    return _combine(g, p, att, basis)


# speedup vs baseline: 1.0121x; 1.0092x over previous
import functools

import jax
import jax.numpy as jnp
from jax import lax
from jax.experimental import pallas as pl
from jax.experimental.pallas import tpu as pltpu
from jax.experimental.pallas import tpu_sc as plsc

N = 10000
D = 128
E = 80000
R = 4
NB = 2

_TILES = 16
_CORES = 2
_REL_PER_CORE = R // _CORES
CH = 128
EPT = E // _TILES
NCH = -(-EPT // CH)
SLOTS = NCH * CH
TAIL = (EPT // 16) * 16
_NBUF = 4
ACC_ROWS = 10240
STRIPE = ACC_ROWS // _TILES
BN = 2000


def _sc_segment_sums(x_bf, e0, e1, e2, e3, zeros_stripe):
    mesh = plsc.VectorSubcoreMesh(core_axis_name="c", subcore_axis_name="s")

    @functools.partial(
        pl.kernel,
        mesh=mesh,
        out_type=jax.ShapeDtypeStruct((R * _TILES, STRIPE, D), jnp.bfloat16),
        compiler_params=pltpu.CompilerParams(use_tc_tiling_on_sc=False),
        scratch_types=[
            pltpu.VMEM((SLOTS,), jnp.int32),
            pltpu.VMEM((SLOTS,), jnp.int32),
            pltpu.VMEM((_NBUF, CH, D), jnp.bfloat16),
            pltpu.VMEM_SHARED((ACC_ROWS, D), jnp.bfloat16),
            pltpu.VMEM_SHARED((N, D), jnp.bfloat16),
        ]
        + [pltpu.SemaphoreType.DMA] * (2 * _NBUF),
    )
    def body(x_hbm, e0_hbm, e1_hbm, e2_hbm, e3_hbm, zero_hbm, out_hbm, src_v, dst_v, rows_v, acc, xs, *sems):
        e_refs = (e0_hbm, e1_hbm, e2_hbm, e3_hbm)
        gsem = sems[:_NBUF]
        ssem = sems[_NBUF:]
        c = lax.axis_index("c")
        s = lax.axis_index("s")

        def src_idx(i):
            return src_v.at[pl.ds(i * CH, CH)]

        def dst_idx(i):
            return dst_v.at[pl.ds(i * CH, CH)]

        def wait_gather(i, b):
            pltpu.make_async_copy(xs.at[src_idx(i)], rows_v.at[b], gsem[b]).wait()

        def wait_scatter(i, b):
            pltpu.make_async_copy(rows_v.at[b], acc.at[dst_idx(i)], ssem[b]).wait()

        pltpu.sync_copy(
            x_hbm.at[pl.ds(s * (N // _TILES), N // _TILES)],
            xs.at[pl.ds(s * (N // _TILES), N // _TILES)],
        )

        for j in range(_REL_PER_CORE):
            rel = c * _REL_PER_CORE + j
            blk = rel * _TILES + s
            pltpu.sync_copy(zero_hbm, acc.at[pl.ds(s * STRIPE, STRIPE)])
            for q in range((SLOTS - TAIL) // 16):
                src_v[pl.ds(TAIL + 16 * q, 16)] = jnp.zeros((16,), jnp.int32)
                dst_v[pl.ds(TAIL + 16 * q, 16)] = jnp.full((16,), N, jnp.int32)
            for cc in range(_CORES):
                @pl.when(c == cc)
                def _stage_idx():
                    e_hbm = e_refs[cc * _REL_PER_CORE + j]
                    pltpu.sync_copy(e_hbm.at[0].at[pl.ds(s * EPT, EPT)], src_v.at[pl.ds(0, EPT)])
                    pltpu.sync_copy(e_hbm.at[1].at[pl.ds(s * EPT, EPT)], dst_v.at[pl.ds(0, EPT)])
            plsc.subcore_barrier()

            for b in range(_NBUF - 1):
                pltpu.async_copy(xs.at[src_idx(b)], rows_v.at[b], gsem[b])

            def step(k, carry):
                for b in range(_NBUF):
                    i = _NBUF * k + b
                    wait_gather(i, b)
                    pltpu.async_copy(rows_v.at[b], acc.at[dst_idx(i)], ssem[b], add=True)
                    b2 = (b + _NBUF - 1) % _NBUF

                    @pl.when(i + _NBUF - 1 < NCH)
                    def _issue():
                        if b == 0:
                            @pl.when(k > 0)
                            def _w():
                                wait_scatter(i - 1, b2)
                        else:
                            wait_scatter(i - 1, b2)
                        pltpu.async_copy(xs.at[src_idx(i + _NBUF - 1)], rows_v.at[b2], gsem[b2])
                return carry

            lax.fori_loop(0, NCH // _NBUF, step, 0)
            for b in range(_NBUF):
                wait_scatter(NCH - _NBUF + b, b)
            plsc.subcore_barrier()
            pltpu.sync_copy(acc.at[pl.ds(s * STRIPE, STRIPE)], out_hbm.at[blk])

    return body(x_bf, e0, e1, e2, e3, zeros_stripe)


def _combine_body(g_ref, x_ref, att_ref, basis_ref, root_ref, o_ref):
    acc = jnp.dot(x_ref[...], root_ref[...], preferred_element_type=jnp.float32)
    gf = g_ref[...].astype(jnp.float32)
    g0, g1, g2, g3 = gf[0], gf[1], gf[2], gf[3]
    m0 = att_ref[0, 0] * g0 + att_ref[1, 0] * g1 + att_ref[2, 0] * g2 + att_ref[3, 0] * g3
    m1 = att_ref[0, 1] * g0 + att_ref[1, 1] * g1 + att_ref[2, 1] * g2 + att_ref[3, 1] * g3
    acc = acc + 0.25 * (
        jnp.dot(m0, basis_ref[0], preferred_element_type=jnp.float32)
        + jnp.dot(m1, basis_ref[1], preferred_element_type=jnp.float32)
    )
    o_ref[...] = acc


def _combine(g, x, att, basis, root):
    return pl.pallas_call(
        _combine_body,
        grid=(N // BN,),
        in_specs=[
            pl.BlockSpec((R, BN, D), lambda i: (0, i, 0)),
            pl.BlockSpec((BN, D), lambda i: (i, 0)),
            pl.BlockSpec(memory_space=pltpu.SMEM),
            pl.BlockSpec((NB, D, D), lambda i: (0, 0, 0)),
            pl.BlockSpec((D, D), lambda i: (0, 0)),
        ],
        out_specs=pl.BlockSpec((BN, D), lambda i: (i, 0)),
        out_shape=jax.ShapeDtypeStruct((N, D), jnp.float32),
    )(g, x, att, basis, root)


def kernel(x, edge_index_0, edge_index_1, edge_index_2, edge_index_3, dest, att, basis, root):
    del dest
    zeros_stripe = jnp.zeros((STRIPE, D), jnp.bfloat16)

    g = _sc_segment_sums(
        x.astype(jnp.bfloat16),
        edge_index_0, edge_index_1, edge_index_2, edge_index_3,
        zeros_stripe,
    )
    g = g.reshape(R, ACC_ROWS, D)
    return _combine(g, x, att, basis, root)
